# baseline (speedup 1.0000x reference)
import functools

import jax
import jax.numpy as jnp
import numpy as np
from jax import lax
from jax.experimental import pallas as pl
from jax.experimental.pallas import tpu as pltpu
from jax.experimental.pallas import tpu_sc as plsc

B, T, C = 8, 200, 32
BW = 8
NC, NS = 2, 16
NEG = np.float32(-np.inf)

_DNUMS = lax.GatherDimensionNumbers(
    offset_dims=(), collapsed_slice_dims=(0,), start_index_map=(0,))


def _gat(x, idx):
    return lax.gather(x, idx.reshape(16, 1), _DNUMS, (1,),
                      mode=lax.GatherScatterMode.PROMISE_IN_BOUNDS)


def _decode_body(logp_hbm, dec_hbm, len_hbm, sc_hbm,
                 lp_v, tc0_v, bps_v, path_v, dec_v,
                 stage_v, big_v, lens_v, scs_v, shared_ls):
    wid = lax.axis_index("c") * NS + lax.axis_index("s")

    @pl.when(wid < B)
    def _():
        b = wid
        pltpu.sync_copy(logp_hbm.at[b], lp_v)

        lane = lax.iota(jnp.int32, 16)
        i_pat = jnp.where(lane < 8, 0,
                          jnp.where(lane < 12, 1, jnp.where(lane < 14, 2, 3)))
        j_pat = lane - jnp.where(lane < 8, 0,
                                 jnp.where(lane < 12, 8,
                                           jnp.where(lane < 14, 12, 14)))
        g_pat = jnp.where(jnp.logical_and(lane >= 8, lane < 12), lane - 4, 0)
        zero16 = lane * 0
        ipat32 = i_pat * 32
        gpat32 = g_pat * 32
        neg16 = jnp.full((16,), NEG, jnp.float32)

        def presort_row(t):
            k0 = lp_v[t, pl.ds(0, 16)]
            k1 = lp_v[t, pl.ds(16, 16)]
            k0s, v0s = plsc.sort_key_val(k0, lane, descending=True)
            k1s, v1s = plsc.sort_key_val(k1, lane + 16, descending=True)
            sh8 = jnp.maximum(lane - 8, 0)
            km = jnp.where(lane < 8, k0s, _gat(k1s, sh8))
            vm = jnp.where(lane < 8, v0s, _gat(v1s, sh8))
            return plsc.sort_key_val(km, vm, descending=True)

        def step(t, carry):
            acc, tk, tc = carry
            kA = _gat(acc, i_pat) + _gat(tk, j_pat)
            vA = ipat32 + _gat(tc, j_pat)
            kAs, vAs = plsc.sort_key_val(kA, vA, descending=True)
            kB = _gat(acc, g_pat) + _gat(tk, zero16)
            vB = gpat32 + _gat(tc, zero16)
            kM = jnp.where(lane < 8, kAs, jnp.where(lane < 12, kB, neg16))
            vM = jnp.where(lane < 8, vAs, vB)
            kMs, vMs = plsc.sort_key_val(kM, vM, descending=True)
            bps_v[t, :] = vMs
            tkn, tcn = presort_row(jnp.minimum(t + 1, T - 1))
            return (kMs, tkn, tcn)

        with jax.named_scope("beamscan"):
            tk0, tc0 = presort_row(0)
            tc0_v[:] = tc0
            tk1, tc1 = presort_row(1)
            acc, _, _ = lax.fori_loop(1, T, step, (tk0, tk1, tc1))

        nblk = (T + 15) // 16

        def bt_block(i, evec):
            blk = (nblk - 1) - i
            v = zero16
            for k in range(15, -1, -1):
                t = blk * 16 + k
                tt = jnp.minimum(t, T - 1)
                row = bps_v[tt, :]
                p = _gat(row, evec)
                sym = jnp.bitwise_and(p, 31)
                if k == 0:
                    p0 = _gat(tc0_v[:], evec)
                    sym = jnp.where(t == 0, p0, sym)
                valid = t < T
                v = jnp.where(jnp.logical_and(lane == k, valid), sym, v)
                evec = jnp.where(jnp.logical_and(valid, t > 0),
                                 jnp.right_shift(p, 5), evec)
            path_v[pl.ds(blk * 16, 16)] = v
            return evec

        with jax.named_scope("backtrack"):
            lax.fori_loop(0, nblk, bt_block, zero16)

        def initdec(i, _):
            dec_v[pl.ds(i * 16, 16)] = jnp.full((16,), -1, jnp.int32)
            return 0

        lax.fori_loop(0, 16, initdec, 0)

        last16 = zero16 + 15

        def collapse(blk, carry):
            pos, mprev = carry
            t0 = blk * 16
            cc = path_v[pl.ds(t0, 16)]
            nb = jnp.where(cc != 0, lane + t0, -1)
            cm = plsc.cummax(nb)
            tot = jnp.maximum(cm, mprev)
            excl = jnp.where(lane == 0, mprev,
                             _gat(tot, jnp.maximum(lane - 1, 0)))
            lastv = plsc.load_gather(path_v, [jnp.maximum(excl, 0)])
            lastc = jnp.where(excl < 0, -1, lastv)
            keep = jnp.logical_and(cc != 0, cc != lastc)
            csum = plsc.cumsum(keep.astype(jnp.int32))
            idx = pos + csum - 1
            plsc.store_scatter(dec_v, [jnp.where(keep, idx, 255)], cc,
                               mask=keep)
            return (pos + _gat(csum, last16), _gat(tot, last16))

        with jax.named_scope("collapse"):
            pos, _ = lax.fori_loop(0, nblk, collapse,
                                   (zero16, zero16 - 1))

        score_bits = plsc.bitcast(_gat(acc, zero16), jnp.int32)
        pltpu.sync_copy(dec_v, dec_hbm.at[b])
        stage_v[:] = jnp.where(lane == 0, pos,
                               jnp.where(lane == 1, score_bits, 0))
        pltpu.sync_copy(stage_v, shared_ls.at[pl.ds(b * 16, 16)])

    plsc.subcore_barrier()

    @pl.when(wid == 0)
    def _():
        lane = lax.iota(jnp.int32, 16)
        pltpu.sync_copy(shared_ls, big_v)
        lens_v[:] = plsc.load_gather(big_v, [lane * 16])
        scs_v[:] = plsc.bitcast(
            plsc.load_gather(big_v, [lane * 16 + 1]), jnp.float32)
        pltpu.sync_copy(lens_v.at[pl.ds(0, B)], len_hbm)
        pltpu.sync_copy(scs_v.at[pl.ds(0, B)], sc_hbm)


@jax.jit
def _sc_decode(logp):
    mesh = plsc.VectorSubcoreMesh(core_axis_name="c", subcore_axis_name="s",
                                  num_cores=1, num_subcores=NS)
    fn = pl.kernel(
        _decode_body,
        out_type=(
            jax.ShapeDtypeStruct((B, 256), jnp.int32),
            jax.ShapeDtypeStruct((B,), jnp.int32),
            jax.ShapeDtypeStruct((B,), jnp.float32),
        ),
        mesh=mesh,
        compiler_params=pltpu.CompilerParams(needs_layout_passes=False),
        scratch_types=[
            pltpu.VMEM((T, C), jnp.float32),
            pltpu.VMEM((16,), jnp.int32),
            pltpu.VMEM((T, 16), jnp.int32),
            pltpu.VMEM((256,), jnp.int32),
            pltpu.VMEM((256,), jnp.int32),
            pltpu.VMEM((16,), jnp.int32),
            pltpu.VMEM((256,), jnp.int32),
            pltpu.VMEM((16,), jnp.int32),
            pltpu.VMEM((16,), jnp.float32),
            pltpu.VMEM_SHARED((256,), jnp.int32),
        ],
    )
    return fn(logp)


def kernel(logits):
    logp = jax.nn.log_softmax(logits.astype(jnp.float32), axis=-1)
    dec_pad, lengths, scores = _sc_decode(logp)
    return dec_pad[:, :T], lengths, scores


# speedup vs baseline: 1.1190x; 1.1190x over previous
import functools

import jax
import jax.numpy as jnp
import numpy as np
from jax import lax
from jax.experimental import pallas as pl
from jax.experimental.pallas import tpu as pltpu
from jax.experimental.pallas import tpu_sc as plsc

B, T, C = 8, 200, 32
BW = 8
NC, NS = 2, 16
NEG = np.float32(-np.inf)

_DNUMS = lax.GatherDimensionNumbers(
    offset_dims=(), collapsed_slice_dims=(0,), start_index_map=(0,))


def _gat(x, idx):
    return lax.gather(x, idx.reshape(16, 1), _DNUMS, (1,),
                      mode=lax.GatherScatterMode.PROMISE_IN_BOUNDS)


def _decode_body(logp_hbm, dec_hbm, len_hbm, sc_hbm,
                 lp_v, topk_v, topc_v, tc0_v, bps_v, path_v, dec_v,
                 stage_v, big_v, lens_v, scs_v, shared_ls):
    wid = lax.axis_index("c") * NS + lax.axis_index("s")

    @pl.when(wid < B)
    def _():
        b = wid
        pltpu.sync_copy(logp_hbm.at[b], lp_v)

        lane = lax.iota(jnp.int32, 16)
        i_pat = jnp.where(lane < 8, 0,
                          jnp.where(lane < 12, 1, jnp.where(lane < 14, 2, 3)))
        j_pat = lane - jnp.where(lane < 8, 0,
                                 jnp.where(lane < 12, 8,
                                           jnp.where(lane < 14, 12, 14)))
        g_pat = jnp.where(jnp.logical_and(lane >= 8, lane < 12), lane - 4, 0)
        zero16 = lane * 0
        ipat32 = i_pat * 32
        gpat32 = g_pat * 32
        neg16 = jnp.full((16,), NEG, jnp.float32)

        def presort(t, _):
            k0 = lp_v[t, pl.ds(0, 16)]
            k1 = lp_v[t, pl.ds(16, 16)]
            k0s, v0s = plsc.sort_key_val(k0, lane, descending=True)
            k1s, v1s = plsc.sort_key_val(k1, lane + 16, descending=True)
            sh8 = jnp.maximum(lane - 8, 0)
            km = jnp.where(lane < 8, k0s, _gat(k1s, sh8))
            vm = jnp.where(lane < 8, v0s, _gat(v1s, sh8))
            kms, vms = plsc.sort_key_val(km, vm, descending=True)
            topk_v[t, :] = kms
            topc_v[t, :] = vms
            return 0

        with jax.named_scope("presort"):
            lax.fori_loop(0, T, presort, 0)
        tc0_v[:] = topc_v[0, :]

        def step(t, acc):
            tk = topk_v[t, :]
            tc = topc_v[t, :]
            kA = _gat(acc, i_pat) + _gat(tk, j_pat)
            vA = ipat32 + _gat(tc, j_pat)
            kAs, vAs = plsc.sort_key_val(kA, vA, descending=True)
            kB = _gat(acc, g_pat) + _gat(tk, zero16)
            vB = gpat32 + _gat(tc, zero16)
            kM = jnp.where(lane < 8, kAs, jnp.where(lane < 12, kB, neg16))
            vM = jnp.where(lane < 8, vAs, vB)
            kMs, vMs = plsc.sort_key_val(kM, vM, descending=True)
            bps_v[t, :] = vMs
            return kMs

        with jax.named_scope("beamscan"):
            acc = lax.fori_loop(1, T, step, topk_v[0, :])

        nblk = (T + 15) // 16

        def bt_block(i, evec):
            blk = (nblk - 1) - i
            v = zero16
            for k in range(15, -1, -1):
                t = blk * 16 + k
                tt = jnp.minimum(t, T - 1)
                row = bps_v[tt, :]
                p = _gat(row, evec)
                sym = jnp.bitwise_and(p, 31)
                if k == 0:
                    p0 = _gat(tc0_v[:], evec)
                    sym = jnp.where(t == 0, p0, sym)
                valid = t < T
                v = jnp.where(jnp.logical_and(lane == k, valid), sym, v)
                evec = jnp.where(jnp.logical_and(valid, t > 0),
                                 jnp.right_shift(p, 5), evec)
            path_v[pl.ds(blk * 16, 16)] = v
            return evec

        with jax.named_scope("backtrack"):
            lax.fori_loop(0, nblk, bt_block, zero16)

        def initdec(i, _):
            dec_v[pl.ds(i * 16, 16)] = jnp.full((16,), -1, jnp.int32)
            return 0

        lax.fori_loop(0, 16, initdec, 0)

        last16 = zero16 + 15

        def collapse(blk, carry):
            pos, mprev = carry
            t0 = blk * 16
            cc = path_v[pl.ds(t0, 16)]
            nb = jnp.where(cc != 0, lane + t0, -1)
            cm = plsc.cummax(nb)
            tot = jnp.maximum(cm, mprev)
            excl = jnp.where(lane == 0, mprev,
                             _gat(tot, jnp.maximum(lane - 1, 0)))
            lastv = plsc.load_gather(path_v, [jnp.maximum(excl, 0)])
            lastc = jnp.where(excl < 0, -1, lastv)
            keep = jnp.logical_and(cc != 0, cc != lastc)
            csum = plsc.cumsum(keep.astype(jnp.int32))
            idx = pos + csum - 1
            plsc.store_scatter(dec_v, [jnp.where(keep, idx, 255)], cc,
                               mask=keep)
            return (pos + _gat(csum, last16), _gat(tot, last16))

        with jax.named_scope("collapse"):
            pos, _ = lax.fori_loop(0, nblk, collapse,
                                   (zero16, zero16 - 1))

        score_bits = plsc.bitcast(_gat(acc, zero16), jnp.int32)
        pltpu.sync_copy(dec_v, dec_hbm.at[b])
        stage_v[:] = jnp.where(lane == 0, pos,
                               jnp.where(lane == 1, score_bits, 0))
        pltpu.sync_copy(stage_v, shared_ls.at[pl.ds(b * 16, 16)])

    plsc.subcore_barrier()

    @pl.when(wid == 0)
    def _():
        lane = lax.iota(jnp.int32, 16)
        pltpu.sync_copy(shared_ls, big_v)
        lens_v[:] = plsc.load_gather(big_v, [lane * 16])
        scs_v[:] = plsc.bitcast(
            plsc.load_gather(big_v, [lane * 16 + 1]), jnp.float32)
        pltpu.sync_copy(lens_v.at[pl.ds(0, B)], len_hbm)
        pltpu.sync_copy(scs_v.at[pl.ds(0, B)], sc_hbm)


@jax.jit
def _sc_decode(logp):
    mesh = plsc.VectorSubcoreMesh(core_axis_name="c", subcore_axis_name="s",
                                  num_cores=1, num_subcores=NS)
    fn = pl.kernel(
        _decode_body,
        out_type=(
            jax.ShapeDtypeStruct((B, 256), jnp.int32),
            jax.ShapeDtypeStruct((B,), jnp.int32),
            jax.ShapeDtypeStruct((B,), jnp.float32),
        ),
        mesh=mesh,
        compiler_params=pltpu.CompilerParams(needs_layout_passes=False),
        scratch_types=[
            pltpu.VMEM((T, C), jnp.float32),
            pltpu.VMEM((T, 16), jnp.float32),
            pltpu.VMEM((T, 16), jnp.int32),
            pltpu.VMEM((16,), jnp.int32),
            pltpu.VMEM((T, 16), jnp.int32),
            pltpu.VMEM((256,), jnp.int32),
            pltpu.VMEM((256,), jnp.int32),
            pltpu.VMEM((16,), jnp.int32),
            pltpu.VMEM((256,), jnp.int32),
            pltpu.VMEM((16,), jnp.int32),
            pltpu.VMEM((16,), jnp.float32),
            pltpu.VMEM_SHARED((256,), jnp.int32),
        ],
    )
    return fn(logp)


def kernel(logits):
    logp = jax.nn.log_softmax(logits.astype(jnp.float32), axis=-1)
    dec_pad, lengths, scores = _sc_decode(logp)
    return dec_pad[:, :T], lengths, scores


# speedup vs baseline: 1.1239x; 1.0043x over previous
import jax
import jax.numpy as jnp
import numpy as np
from jax import lax
from jax.experimental import pallas as pl
from jax.experimental.pallas import tpu as pltpu
from jax.experimental.pallas import tpu_sc as plsc

B, T, C = 8, 200, 32
NS = 16
NEG = np.float32(-np.inf)

_DNUMS = lax.GatherDimensionNumbers(
    offset_dims=(), collapsed_slice_dims=(0,), start_index_map=(0,))


def _gat(x, idx):
    return lax.gather(x, idx.reshape(16, 1), _DNUMS, (1,),
                      mode=lax.GatherScatterMode.PROMISE_IN_BOUNDS)


def _decode_body(logp_hbm, dec_hbm, len_hbm, sc_hbm,
                 lp_v, topk_v, topc_v, tc0_v, bps_v, path_v, dec_v,
                 stage_v, big_v, lens_v, scs_v, shared_ls):
    wid = lax.axis_index("c") * NS + lax.axis_index("s")

    @pl.when(wid < B)
    def _():
        b = wid
        pltpu.sync_copy(logp_hbm.at[b], lp_v)

        lane = lax.iota(jnp.int32, 16)
        i_pat = jnp.where(lane < 8, 0,
                          jnp.where(lane < 12, 1, jnp.where(lane < 14, 2, 3)))
        j_pat = lane - jnp.where(lane < 8, 0,
                                 jnp.where(lane < 12, 8,
                                           jnp.where(lane < 14, 12, 14)))
        g_pat = jnp.where(jnp.logical_and(lane >= 8, lane < 12), lane - 4, 0)
        zero16 = lane * 0
        ipat32 = i_pat * 32
        gpat32 = g_pat * 32
        neg16 = jnp.full((16,), NEG, jnp.float32)

        def presort(t, _):
            k0 = lp_v[t, pl.ds(0, 16)]
            k1 = lp_v[t, pl.ds(16, 16)]
            k0s, v0s = plsc.sort_key_val(k0, lane, descending=True)
            k1s, v1s = plsc.sort_key_val(k1, lane + 16, descending=True)
            sh8 = jnp.maximum(lane - 8, 0)
            km = jnp.where(lane < 8, k0s, _gat(k1s, sh8))
            vm = jnp.where(lane < 8, v0s, _gat(v1s, sh8))
            kms, vms = plsc.sort_key_val(km, vm, descending=True)
            topk_v[t, :] = kms
            topc_v[t, :] = vms
            return 0

        with jax.named_scope("presort"):
            lax.fori_loop(0, T, presort, 0)
        tc0_v[:] = topc_v[0, :]

        def step(t, acc):
            tk = topk_v[t, :]
            tc = topc_v[t, :]
            kA = _gat(acc, i_pat) + _gat(tk, j_pat)
            vA = ipat32 + _gat(tc, j_pat)
            kAs, vAs = plsc.sort_key_val(kA, vA, descending=True)
            kB = _gat(acc, g_pat) + _gat(tk, zero16)
            vB = gpat32 + _gat(tc, zero16)
            kM = jnp.where(lane < 8, kAs, jnp.where(lane < 12, kB, neg16))
            vM = jnp.where(lane < 8, vAs, vB)
            kMs, vMs = plsc.sort_key_val(kM, vM, descending=True)
            bps_v[t, :] = vMs
            return kMs

        with jax.named_scope("beamscan"):
            acc = lax.fori_loop(1, T, step, topk_v[0, :])

        nblk = (T + 15) // 16

        def bt_block(i, evec):
            blk = (nblk - 1) - i
            v = zero16
            for k in range(15, -1, -1):
                t = blk * 16 + k
                tt = jnp.minimum(t, T - 1)
                row = bps_v[tt, :]
                p = _gat(row, evec)
                sym = jnp.bitwise_and(p, 31)
                if k == 0:
                    p0 = _gat(tc0_v[:], evec)
                    sym = jnp.where(t == 0, p0, sym)
                valid = t < T
                v = jnp.where(jnp.logical_and(lane == k, valid), sym, v)
                evec = jnp.where(jnp.logical_and(valid, t > 0),
                                 jnp.right_shift(p, 5), evec)
            path_v[pl.ds(blk * 16, 16)] = v
            return evec

        with jax.named_scope("backtrack"):
            lax.fori_loop(0, nblk, bt_block, zero16)

        def initdec(i, _):
            dec_v[pl.ds(i * 16, 16)] = jnp.full((16,), -1, jnp.int32)
            return 0

        lax.fori_loop(0, 16, initdec, 0)

        last16 = zero16 + 15

        def collapse(blk, carry):
            pos, mprev = carry
            t0 = blk * 16
            cc = path_v[pl.ds(t0, 16)]
            nb = jnp.where(cc != 0, lane + t0, -1)
            cm = plsc.cummax(nb)
            tot = jnp.maximum(cm, mprev)
            excl = jnp.where(lane == 0, mprev,
                             _gat(tot, jnp.maximum(lane - 1, 0)))
            lastv = plsc.load_gather(path_v, [jnp.maximum(excl, 0)])
            lastc = jnp.where(excl < 0, -1, lastv)
            keep = jnp.logical_and(cc != 0, cc != lastc)
            csum = plsc.cumsum(keep.astype(jnp.int32))
            idx = pos + csum - 1
            plsc.store_scatter(dec_v, [jnp.where(keep, idx, 255)], cc,
                               mask=keep)
            return (pos + _gat(csum, last16), _gat(tot, last16))

        with jax.named_scope("collapse"):
            pos, _ = lax.fori_loop(0, nblk, collapse,
                                   (zero16, zero16 - 1))

        score_bits = plsc.bitcast(_gat(acc, zero16), jnp.int32)
        pltpu.sync_copy(dec_v, dec_hbm.at[b])
        stage_v[:] = jnp.where(lane == 0, pos,
                               jnp.where(lane == 1, score_bits, 0))
        pltpu.sync_copy(stage_v, shared_ls.at[pl.ds(b * 16, 16)])

    plsc.subcore_barrier()

    @pl.when(wid == 0)
    def _():
        lane = lax.iota(jnp.int32, 16)
        pltpu.sync_copy(shared_ls, big_v)
        lens_v[:] = plsc.load_gather(big_v, [lane * 16])
        scs_v[:] = plsc.bitcast(
            plsc.load_gather(big_v, [lane * 16 + 1]), jnp.float32)
        pltpu.sync_copy(lens_v.at[pl.ds(0, B)], len_hbm)
        pltpu.sync_copy(scs_v.at[pl.ds(0, B)], sc_hbm)


@jax.jit
def _sc_decode(logp):
    mesh = plsc.VectorSubcoreMesh(core_axis_name="c", subcore_axis_name="s",
                                  num_cores=1, num_subcores=NS)
    fn = pl.kernel(
        _decode_body,
        out_type=(
            jax.ShapeDtypeStruct((B, 256), jnp.int32),
            jax.ShapeDtypeStruct((B,), jnp.int32),
            jax.ShapeDtypeStruct((B,), jnp.float32),
        ),
        mesh=mesh,
        compiler_params=pltpu.CompilerParams(needs_layout_passes=False),
        scratch_types=[
            pltpu.VMEM((T, C), jnp.float32),
            pltpu.VMEM((T, 16), jnp.float32),
            pltpu.VMEM((T, 16), jnp.int32),
            pltpu.VMEM((16,), jnp.int32),
            pltpu.VMEM((T, 16), jnp.int32),
            pltpu.VMEM((256,), jnp.int32),
            pltpu.VMEM((256,), jnp.int32),
            pltpu.VMEM((16,), jnp.int32),
            pltpu.VMEM((256,), jnp.int32),
            pltpu.VMEM((16,), jnp.int32),
            pltpu.VMEM((16,), jnp.float32),
            pltpu.VMEM_SHARED((256,), jnp.int32),
        ],
    )
    return fn(logp)


def kernel(logits):
    logp = jax.nn.log_softmax(logits.astype(jnp.float32), axis=-1)
    dec_pad, lengths, scores = _sc_decode(logp)
    return dec_pad[:, :T], lengths, scores
